# baseline (speedup 1.0000x reference)
import jax
import jax.numpy as jnp
from jax.experimental import pallas as pl
from jax.experimental.pallas import tpu as pltpu

D_OUT = 4096
D_IN = 4096
RANK = 64
N_CODES = 16

BM = 1024
BN = 4096


def _dequant(idx, codebook_row):
    out = jnp.full(idx.shape, codebook_row[0, 0], jnp.float32)
    for p in range(1, N_CODES):
        out = jnp.where(idx == p, codebook_row[0, p], out)
    return out


def _fused_kernel(a_idx_ref, b_idx_ref, ca_ref, cb_ref, out_ref,
                  a_deq_ref, b_deq_ref):
    i = pl.program_id(0)
    j = pl.program_id(1)

    @pl.when((i == 0) & (j == 0))
    def _():
        a_deq_ref[...] = _dequant(a_idx_ref[...], ca_ref[...])
        b_deq_ref[...] = _dequant(b_idx_ref[...], cb_ref[...])

    a = a_deq_ref[pl.ds(i * BM, BM), :]
    b = b_deq_ref[:, pl.ds(j * BN, BN)]
    out_ref[...] = jax.lax.dot_general(
        a, b, (((1,), (0,)), ((), ())),
        preferred_element_type=jnp.float32,
        precision=jax.lax.Precision.DEFAULT,
    )


def kernel(A_assignments, B_assignments, A_codebook, B_codebook):
    ca = A_codebook.reshape(1, N_CODES).astype(jnp.float32)
    cb = B_codebook.reshape(1, N_CODES).astype(jnp.float32)
    grid = (D_OUT // BM, D_IN // BN)
    return pl.pallas_call(
        _fused_kernel,
        grid=grid,
        in_specs=[
            pl.BlockSpec((D_OUT, RANK), lambda i, j: (0, 0)),
            pl.BlockSpec((RANK, D_IN), lambda i, j: (0, 0)),
            pl.BlockSpec((1, N_CODES), lambda i, j: (0, 0)),
            pl.BlockSpec((1, N_CODES), lambda i, j: (0, 0)),
        ],
        out_specs=pl.BlockSpec((BM, BN), lambda i, j: (i, j)),
        out_shape=jax.ShapeDtypeStruct((D_OUT, D_IN), jnp.float32),
        scratch_shapes=[
            pltpu.VMEM((D_OUT, RANK), jnp.float32),
            pltpu.VMEM((RANK, D_IN), jnp.float32),
        ],
        compiler_params=pltpu.CompilerParams(
            dimension_semantics=("arbitrary", "arbitrary"),
        ),
    )(A_assignments, B_assignments, ca, cb)


# speedup vs baseline: 1.0506x; 1.0506x over previous
import jax
import jax.numpy as jnp
from jax.experimental import pallas as pl
from jax.experimental.pallas import tpu as pltpu

D_OUT = 4096
D_IN = 4096
RANK = 64
N_CODES = 16

BM = 512


def _dequant(idx, codebook_row):
    out = jnp.full(idx.shape, codebook_row[0, 0], jnp.float32)
    for p in range(1, N_CODES):
        out = jnp.where(idx == p, codebook_row[0, p], out)
    return out


def _out_copy(obuf_ref, hbm_out_ref, sem, step, slot):
    return pltpu.make_async_copy(
        obuf_ref.at[slot],
        hbm_out_ref.at[pl.ds(step * BM, BM), :],
        sem.at[slot],
    )


def _fused_kernel(a_idx_ref, b_idx_ref, ca_ref, cb_ref, hbm_out_ref,
                  a_deq_ref, b_deq_ref, obuf_ref, sem):
    i = pl.program_id(0)
    n = pl.num_programs(0)
    slot = jax.lax.rem(i, 2)

    @pl.when(i == 0)
    def _():
        a_deq_ref[...] = _dequant(a_idx_ref[...], ca_ref[...])
        b_deq_ref[...] = _dequant(b_idx_ref[...], cb_ref[...])

    @pl.when(i >= 2)
    def _():
        _out_copy(obuf_ref, hbm_out_ref, sem, i - 2, slot).wait()

    a = a_deq_ref[pl.ds(i * BM, BM), :]
    obuf_ref[slot] = jax.lax.dot_general(
        a, b_deq_ref[...], (((1,), (0,)), ((), ())),
        preferred_element_type=jnp.float32,
        precision=jax.lax.Precision.DEFAULT,
    )
    _out_copy(obuf_ref, hbm_out_ref, sem, i, slot).start()

    @pl.when(i == n - 1)
    def _():
        _out_copy(obuf_ref, hbm_out_ref, sem, i - 1, 1 - slot).wait()
        _out_copy(obuf_ref, hbm_out_ref, sem, i, slot).wait()


def kernel(A_assignments, B_assignments, A_codebook, B_codebook):
    ca = A_codebook.reshape(1, N_CODES).astype(jnp.float32)
    cb = B_codebook.reshape(1, N_CODES).astype(jnp.float32)
    return pl.pallas_call(
        _fused_kernel,
        grid=(D_OUT // BM,),
        in_specs=[
            pl.BlockSpec((D_OUT, RANK), lambda i: (0, 0)),
            pl.BlockSpec((RANK, D_IN), lambda i: (0, 0)),
            pl.BlockSpec((1, N_CODES), lambda i: (0, 0)),
            pl.BlockSpec((1, N_CODES), lambda i: (0, 0)),
        ],
        out_specs=pl.BlockSpec(memory_space=pl.ANY),
        out_shape=jax.ShapeDtypeStruct((D_OUT, D_IN), jnp.float32),
        scratch_shapes=[
            pltpu.VMEM((D_OUT, RANK), jnp.float32),
            pltpu.VMEM((RANK, D_IN), jnp.float32),
            pltpu.VMEM((2, BM, D_IN), jnp.float32),
            pltpu.SemaphoreType.DMA((2,)),
        ],
        compiler_params=pltpu.CompilerParams(
            dimension_semantics=("arbitrary",),
        ),
    )(A_assignments, B_assignments, ca, cb)


# speedup vs baseline: 1.0584x; 1.0074x over previous
import jax
import jax.numpy as jnp
from jax.experimental import pallas as pl
from jax.experimental.pallas import tpu as pltpu

D_OUT = 4096
D_IN = 4096
RANK = 64
N_CODES = 16

BM = 512


def _dequant(idx, codebook_row):
    out = jnp.full(idx.shape, codebook_row[0, 0], jnp.float32)
    for p in range(1, N_CODES):
        out = jnp.where(idx == p, codebook_row[0, p], out)
    return out


def _out_copy(obuf_ref, hbm_out_ref, sem, step, slot):
    return pltpu.make_async_copy(
        obuf_ref.at[slot],
        hbm_out_ref.at[pl.ds(step * BM, BM), :],
        sem.at[slot],
    )


def _fused_kernel(a_idx_hbm, b_idx_hbm, ca_hbm, cb_hbm, hbm_out_ref,
                  a_idx_ref, b_idx_ref, ca_ref, cb_ref,
                  a_deq_ref, b_deq_ref, obuf_ref, sem, in_sem):
    i = pl.program_id(0)
    n = pl.num_programs(0)
    slot = jax.lax.rem(i, 2)

    @pl.when(i == 0)
    def _():
        copies = (
            pltpu.make_async_copy(a_idx_hbm, a_idx_ref, in_sem.at[0]),
            pltpu.make_async_copy(b_idx_hbm, b_idx_ref, in_sem.at[1]),
            pltpu.make_async_copy(ca_hbm, ca_ref, in_sem.at[2]),
            pltpu.make_async_copy(cb_hbm, cb_ref, in_sem.at[3]),
        )
        for c in copies:
            c.start()
        for c in copies:
            c.wait()
        a_deq_ref[...] = _dequant(a_idx_ref[...], ca_ref[...])
        b_deq_ref[...] = _dequant(b_idx_ref[...], cb_ref[...])

    @pl.when(i >= 2)
    def _():
        _out_copy(obuf_ref, hbm_out_ref, sem, i - 2, slot).wait()

    a = a_deq_ref[pl.ds(i * BM, BM), :]
    obuf_ref[slot] = jax.lax.dot_general(
        a, b_deq_ref[...], (((1,), (0,)), ((), ())),
        preferred_element_type=jnp.float32,
        precision=jax.lax.Precision.DEFAULT,
    )
    _out_copy(obuf_ref, hbm_out_ref, sem, i, slot).start()

    @pl.when(i == n - 1)
    def _():
        _out_copy(obuf_ref, hbm_out_ref, sem, i - 1, 1 - slot).wait()
        _out_copy(obuf_ref, hbm_out_ref, sem, i, slot).wait()


def kernel(A_assignments, B_assignments, A_codebook, B_codebook):
    ca = A_codebook.reshape(1, N_CODES).astype(jnp.float32)
    cb = B_codebook.reshape(1, N_CODES).astype(jnp.float32)
    return pl.pallas_call(
        _fused_kernel,
        grid=(D_OUT // BM,),
        in_specs=[
            pl.BlockSpec(memory_space=pl.ANY),
            pl.BlockSpec(memory_space=pl.ANY),
            pl.BlockSpec(memory_space=pl.ANY),
            pl.BlockSpec(memory_space=pl.ANY),
        ],
        out_specs=pl.BlockSpec(memory_space=pl.ANY),
        out_shape=jax.ShapeDtypeStruct((D_OUT, D_IN), jnp.float32),
        scratch_shapes=[
            pltpu.VMEM((D_OUT, RANK), jnp.int32),
            pltpu.VMEM((RANK, D_IN), jnp.int32),
            pltpu.VMEM((1, N_CODES), jnp.float32),
            pltpu.VMEM((1, N_CODES), jnp.float32),
            pltpu.VMEM((D_OUT, RANK), jnp.float32),
            pltpu.VMEM((RANK, D_IN), jnp.float32),
            pltpu.VMEM((2, BM, D_IN), jnp.float32),
            pltpu.SemaphoreType.DMA((2,)),
            pltpu.SemaphoreType.DMA((4,)),
        ],
        compiler_params=pltpu.CompilerParams(
            dimension_semantics=("arbitrary",),
        ),
    )(A_assignments, B_assignments, ca, cb)
